# baseline (speedup 1.0000x reference)
import jax
import jax.numpy as jnp
from jax.experimental import pallas as pl
from jax.experimental.pallas import tpu as pltpu

_N = 8192
_BI = 512
_BJ = 1024
_NI = _N // _BI
_NJ = _N // _BJ


def _chamfer_block_kernel(p_ref, t_ref, out_ref, rowacc_ref, colacc_ref,
                          sum_ref):
    i = pl.program_id(0)
    j = pl.program_id(1)

    p = p_ref[...]
    t = t_ref[...]

    px, py, pz = p[:, 0:1], p[:, 1:2], p[:, 2:3]
    tx, ty, tz = t[0:1, :], t[1:2, :], t[2:3, :]
    pn = px * px + py * py + pz * pz
    tn = tx * tx + ty * ty + tz * tz

    bf = jnp.bfloat16
    f32 = jnp.float32
    pbx, pby, pbz = (px.astype(bf).astype(f32), py.astype(bf).astype(f32),
                     pz.astype(bf).astype(f32))
    tbx, tby, tbz = (tx.astype(bf).astype(f32), ty.astype(bf).astype(f32),
                     tz.astype(bf).astype(f32))
    s = pbx * tbx + pby * tby + pbz * tbz
    d = (pn + tn) - 2.0 * s

    rowm = jnp.min(d, axis=1, keepdims=True)
    colm = jnp.min(d, axis=0, keepdims=True)

    @pl.when(j == 0)
    def _():
        rowacc_ref[...] = rowm

    @pl.when(j > 0)
    def _():
        rowacc_ref[...] = jnp.minimum(rowacc_ref[...], rowm)

    @pl.when(i == 0)
    def _():
        colacc_ref[0:1, pl.ds(j * _BJ, _BJ)] = colm

    @pl.when(i > 0)
    def _():
        colacc_ref[0:1, pl.ds(j * _BJ, _BJ)] = jnp.minimum(
            colacc_ref[0:1, pl.ds(j * _BJ, _BJ)], colm)

    @pl.when(jnp.logical_and(i == 0, j == 0))
    def _():
        sum_ref[0, 0] = 0.0

    @pl.when(j == _NJ - 1)
    def _():
        sum_ref[0, 0] += jnp.sum(rowacc_ref[...])

    @pl.when(jnp.logical_and(i == _NI - 1, j == _NJ - 1))
    def _():
        total = sum_ref[0, 0] + jnp.sum(colacc_ref[...])
        out_ref[...] = jnp.full((1, 1), total / (2.0 * _N), jnp.float32)


def _chamfer(pred, target_t, interpret=False):
    return pl.pallas_call(
        _chamfer_block_kernel,
        grid=(_NI, _NJ),
        in_specs=[
            pl.BlockSpec((_BI, 3), lambda i, j: (i, 0)),
            pl.BlockSpec((3, _BJ), lambda i, j: (0, j)),
        ],
        out_specs=pl.BlockSpec((1, 1), lambda i, j: (0, 0)),
        out_shape=jax.ShapeDtypeStruct((1, 1), jnp.float32),
        scratch_shapes=[
            pltpu.VMEM((_BI, 1), jnp.float32),
            pltpu.VMEM((1, _N), jnp.float32),
            pltpu.SMEM((1, 1), jnp.float32),
        ],
        interpret=interpret,
    )(pred, target_t)


@jax.jit
def kernel(pred_positions, target_positions):
    out = _chamfer(pred_positions, target_positions.T)
    return out[0, 0]


# speedup vs baseline: 1.3484x; 1.3484x over previous
import jax
import jax.numpy as jnp
from jax.experimental import pallas as pl
from jax.experimental.pallas import tpu as pltpu

_N = 8192
_BI = 512
_BJ = 1024
_NI = _N // _BI
_NJ = _N // _BJ


def _chamfer_block_kernel(p_ref, t_ref, out_ref, rowacc_ref, colacc_ref,
                          sum_ref):
    i = pl.program_id(0)
    j = pl.program_id(1)

    p = p_ref[...]
    t = t_ref[...]

    bf = jnp.bfloat16
    f32 = jnp.float32
    px, py, pz = p[:, 0:1], p[:, 1:2], p[:, 2:3]
    tx, ty, tz = t[0:1, :], t[1:2, :], t[2:3, :]
    pn = px * px + py * py + pz * pz
    tn = tx * tx + ty * ty + tz * tz
    pnh = pn.astype(bf)
    pnl = (pn - pnh.astype(f32)).astype(bf)
    tnh = tn.astype(bf)
    tnl = (tn - tnh.astype(f32)).astype(bf)
    ones_p = jnp.ones((p.shape[0], 1), bf)
    ones_t = jnp.ones((1, t.shape[1]), bf)
    a = jnp.concatenate(
        [(-2.0 * px.astype(bf).astype(f32)).astype(bf),
         (-2.0 * py.astype(bf).astype(f32)).astype(bf),
         (-2.0 * pz.astype(bf).astype(f32)).astype(bf),
         pnh, pnl, ones_p, ones_p], axis=1)
    b = jnp.concatenate(
        [tx.astype(bf), ty.astype(bf), tz.astype(bf),
         ones_t, ones_t, tnh, tnl], axis=0)
    d = jax.lax.dot_general(a, b, (((1,), (0,)), ((), ())),
                            preferred_element_type=f32)

    rowm = jnp.min(d, axis=1, keepdims=True)
    colm = jnp.min(d, axis=0, keepdims=True)

    @pl.when(j == 0)
    def _():
        rowacc_ref[...] = rowm

    @pl.when(j > 0)
    def _():
        rowacc_ref[...] = jnp.minimum(rowacc_ref[...], rowm)

    @pl.when(i == 0)
    def _():
        colacc_ref[0:1, pl.ds(j * _BJ, _BJ)] = colm

    @pl.when(i > 0)
    def _():
        colacc_ref[0:1, pl.ds(j * _BJ, _BJ)] = jnp.minimum(
            colacc_ref[0:1, pl.ds(j * _BJ, _BJ)], colm)

    @pl.when(jnp.logical_and(i == 0, j == 0))
    def _():
        sum_ref[0, 0] = 0.0

    @pl.when(j == _NJ - 1)
    def _():
        sum_ref[0, 0] += jnp.sum(rowacc_ref[...])

    @pl.when(jnp.logical_and(i == _NI - 1, j == _NJ - 1))
    def _():
        total = sum_ref[0, 0] + jnp.sum(colacc_ref[...])
        out_ref[...] = jnp.full((1, 1), total / (2.0 * _N), jnp.float32)


def _chamfer(pred, target_t, interpret=False):
    return pl.pallas_call(
        _chamfer_block_kernel,
        grid=(_NI, _NJ),
        in_specs=[
            pl.BlockSpec((_BI, 3), lambda i, j: (i, 0)),
            pl.BlockSpec((3, _BJ), lambda i, j: (0, j)),
        ],
        out_specs=pl.BlockSpec((1, 1), lambda i, j: (0, 0)),
        out_shape=jax.ShapeDtypeStruct((1, 1), jnp.float32),
        scratch_shapes=[
            pltpu.VMEM((_BI, 1), jnp.float32),
            pltpu.VMEM((1, _N), jnp.float32),
            pltpu.SMEM((1, 1), jnp.float32),
        ],
        interpret=interpret,
    )(pred, target_t)


@jax.jit
def kernel(pred_positions, target_positions):
    out = _chamfer(pred_positions, target_positions.T)
    return out[0, 0]
